# Initial kernel scaffold; baseline (speedup 1.0000x reference)
import jax
import jax.numpy as jnp
from jax.experimental import pallas as pl


def kernel(x_gene, x_disease, edge_index_g2d, edge_index_d2g, edge_label_index, params):
    raise NotImplementedError("write your pallas kernel here")


# speedup vs baseline: 1.0018x; 1.0018x over previous
import functools

import jax
import jax.numpy as jnp
from jax.experimental import pallas as pl
from jax.experimental.pallas import tpu as pltpu

N_GENE = 5000
N_DIS = 5000
HIDDEN = 256
FF_DIM = 2 * HIDDEN
N_LABEL = 8192
NUM_BLOCKS = 2
NUM_FF_HIDDEN = 3


def _gatv2(x_src, x_dst, edge_index, Wl, Wr, att, bias, num_dst):
    xl = x_src @ Wl
    xr = x_dst @ Wr
    src = edge_index[0]
    dst = edge_index[1]
    e = jax.nn.leaky_relu(xl[src] + xr[dst], negative_slope=0.2)
    logits = e @ att
    m = jax.ops.segment_max(logits, dst, num_segments=num_dst)
    m = jnp.where(jnp.isfinite(m), m, 0.0)
    ex = jnp.exp(logits - m[dst])
    denom = jax.ops.segment_sum(ex, dst, num_segments=num_dst)
    alpha = ex / jnp.maximum(denom[dst], 1e-16)
    out = jax.ops.segment_sum(alpha[:, None] * xl[src], dst, num_segments=num_dst)
    return out + bias


def _bn(x, gamma, beta):
    mu = jnp.mean(x, axis=0)
    var = jnp.var(x, axis=0)
    return (x - mu) / jnp.sqrt(var + 1e-5) * gamma + beta


def _mlp_body(h_ref, w0, b0, w1, b1, w2, b2, w3, b3, wf, bf, out_ref):
    h = h_ref[...]
    h = jax.nn.relu(jnp.dot(h, w0[...], preferred_element_type=jnp.float32) + b0[...])
    h = jax.nn.relu(jnp.dot(h, w1[...], preferred_element_type=jnp.float32) + b1[...])
    h = jax.nn.relu(jnp.dot(h, w2[...], preferred_element_type=jnp.float32) + b2[...])
    h = jax.nn.relu(jnp.dot(h, w3[...], preferred_element_type=jnp.float32) + b3[...])
    out_ref[...] = jnp.dot(h, wf[...], preferred_element_type=jnp.float32) + bf[...]


def _mlp(h, params):
    BLK = 2048
    n = h.shape[0]
    ws = []
    for i in range(NUM_FF_HIDDEN + 1):
        ws += [params['ff_W%d' % i], params['ff_b%d' % i].reshape(1, FF_DIM)]
    ws += [params['ff_Wf'], params['ff_bf'].reshape(1, 1)]
    wspecs = []
    for i in range(NUM_FF_HIDDEN + 1):
        wspecs += [pl.BlockSpec((FF_DIM, FF_DIM), lambda i: (0, 0)),
                   pl.BlockSpec((1, FF_DIM), lambda i: (0, 0))]
    wspecs += [pl.BlockSpec((FF_DIM, 1), lambda i: (0, 0)),
               pl.BlockSpec((1, 1), lambda i: (0, 0))]
    out = pl.pallas_call(
        _mlp_body,
        grid=(n // BLK,),
        in_specs=[pl.BlockSpec((BLK, FF_DIM), lambda i: (i, 0))] + wspecs,
        out_specs=pl.BlockSpec((BLK, 1), lambda i: (i, 0)),
        out_shape=jax.ShapeDtypeStruct((n, 1), jnp.float32),
    )(h, *ws)
    return out.reshape(-1)


def kernel(x_gene, x_disease, edge_index_g2d, edge_index_d2g, edge_label_index, params):
    hg, hd = x_gene, x_disease
    for b in range(NUM_BLOCKS):
        p = params['block%d' % b]
        new_d = _gatv2(hg, hd, edge_index_g2d, p['g2d_Wl'], p['g2d_Wr'], p['g2d_att'], p['g2d_bias'], N_DIS)
        new_g = _gatv2(hd, hg, edge_index_d2g, p['d2g_Wl'], p['d2g_Wr'], p['d2g_att'], p['d2g_bias'], N_GENE)
        hg = jax.nn.relu(_bn(new_g, p['bn_gene_gamma'], p['bn_gene_beta']))
        hd = jax.nn.relu(_bn(new_d, p['bn_disease_gamma'], p['bn_disease_beta']))
    gene_embs = hg[edge_label_index[1]]
    dis_embs = hd[edge_label_index[0]]
    h = jnp.concatenate([gene_embs, dis_embs], axis=1)
    return _mlp(h, params)


# speedup vs baseline: 4.7516x; 4.7430x over previous
import functools

import jax
import jax.numpy as jnp
from jax import lax
from jax.experimental import pallas as pl
from jax.experimental.pallas import tpu as pltpu
from jax.experimental.pallas import tpu_sc as plsc

N_GENE = 5000
N_DIS = 5000
HIDDEN = 256
FF_DIM = 2 * HIDDEN
N_LABEL = 8192
NUM_BLOCKS = 2
NUM_FF_HIDDEN = 3
N_EDGES = 160000

NC = 2
NS = 16
NW = NC * NS

EB = 64
NB = N_EDGES // EB
NPW = (NB + NW - 1) // NW
ACC_ROWS = 5120
DEN_W = 8
NCH = HIDDEN // 16
RPS = ACC_ROWS // NS


def _edge_body(xl_hbm, xr_hbm, src_hbm, dst_hbm, att_hbm,
               num_hbm, den_hbm,
               src_v, dst_v, xl_v, xr_v, den_v, att_v, num_sh, den_sh):
    c = lax.axis_index("c")
    s = lax.axis_index("s")
    wid = s * NC + c

    pltpu.sync_copy(att_hbm, att_v)

    zero16 = jnp.zeros((16,), jnp.float32)

    def _zrow(i, _):
        for cc in range(NCH):
            xl_v[i, cc * 16:(cc + 1) * 16] = zero16
        den_v[i, 0:8] = zero16[0:8]
        return 0

    lax.fori_loop(0, EB, _zrow, 0, unroll=False)

    rbase = s * RPS
    for j in range(RPS // EB):
        pltpu.sync_copy(xl_v, num_sh.at[pl.ds(rbase + j * EB, EB)])
        pltpu.sync_copy(den_v, den_sh.at[pl.ds(rbase + j * EB, EB)])
    plsc.subcore_barrier()

    lane = lax.iota(jnp.int32, 16)
    onehot0 = jnp.where(lane == 0, 1.0, 0.0).astype(jnp.float32)
    perms = [jnp.bitwise_xor(lane, sh) for sh in (8, 4, 2, 1)]

    def _batch(ib, _):
        k = wid + ib * NW

        @pl.when(k < NB)
        def _():
            base = k * EB
            pltpu.sync_copy(src_hbm.at[pl.ds(base, EB)], src_v)
            pltpu.sync_copy(dst_hbm.at[pl.ds(base, EB)], dst_v)
            pltpu.sync_copy(xl_hbm.at[src_v], xl_v)
            pltpu.sync_copy(xr_hbm.at[dst_v], xr_v)

            def _edge(i, _):
                acc = jnp.zeros((16,), jnp.float32)
                for cc in range(NCH):
                    sl = slice(cc * 16, (cc + 1) * 16)
                    v = xl_v[i, sl] + xr_v[i, sl]
                    lr = jnp.maximum(v, 0.0) + 0.2 * jnp.minimum(v, 0.0)
                    acc = acc + att_v[sl] * lr
                for p in perms:
                    acc = acc + acc[p]
                ex = jnp.exp(acc)
                for cc in range(NCH):
                    sl = slice(cc * 16, (cc + 1) * 16)
                    xl_v[i, sl] = ex * xl_v[i, sl]
                den_v[i, 0:8] = (ex * onehot0)[0:8]
                return 0

            lax.fori_loop(0, EB, _edge, 0, unroll=False)
            pltpu.sync_copy(xl_v, num_sh.at[dst_v], add=True)
            pltpu.sync_copy(den_v, den_sh.at[dst_v], add=True)

        return 0

    lax.fori_loop(0, NPW, _batch, 0, unroll=False)
    plsc.subcore_barrier()

    for j in range(RPS // EB):
        pltpu.sync_copy(num_sh.at[pl.ds(rbase + j * EB, EB)],
                        num_hbm.at[c, pl.ds(rbase + j * EB, EB)])
    pltpu.sync_copy(den_sh.at[pl.ds(rbase, RPS)],
                    den_hbm.at[c, pl.ds(rbase, RPS)])


_edge_kernel = functools.partial(
    pl.kernel,
    out_type=(jax.ShapeDtypeStruct((NC, ACC_ROWS, HIDDEN), jnp.float32),
              jax.ShapeDtypeStruct((NC, ACC_ROWS, DEN_W), jnp.float32)),
    mesh=plsc.VectorSubcoreMesh(core_axis_name="c", subcore_axis_name="s"),
    compiler_params=pltpu.CompilerParams(use_tc_tiling_on_sc=False),
    scratch_types=[
        pltpu.VMEM((EB,), jnp.int32),
        pltpu.VMEM((EB,), jnp.int32),
        pltpu.VMEM((EB, HIDDEN), jnp.float32),
        pltpu.VMEM((EB, HIDDEN), jnp.float32),
        pltpu.VMEM((EB, DEN_W), jnp.float32),
        pltpu.VMEM((HIDDEN,), jnp.float32),
        pltpu.VMEM_SHARED((ACC_ROWS, HIDDEN), jnp.float32),
        pltpu.VMEM_SHARED((ACC_ROWS, DEN_W), jnp.float32),
    ],
)(_edge_body)


def _gat_edge_stage(xl, xr, src, dst, att, bias):
    num, den = _edge_kernel(xl, xr, src, dst, att)
    numt = num[0, :N_DIS] + num[1, :N_DIS]
    dent = den[0, :N_DIS, 0] + den[1, :N_DIS, 0]
    return numt / jnp.maximum(dent, 1e-16)[:, None] + bias


def _mm_body(x_ref, w_ref, o_ref):
    o_ref[...] = jnp.dot(x_ref[...], w_ref[...],
                         preferred_element_type=jnp.float32)


def _mm(x, w):
    m, kdim = x.shape
    n = w.shape[1]
    mp = 5120
    xpad = jnp.pad(x, ((0, mp - m), (0, 0)))
    BLK = 640
    out = pl.pallas_call(
        _mm_body,
        grid=(mp // BLK,),
        in_specs=[pl.BlockSpec((BLK, kdim), lambda i: (i, 0)),
                  pl.BlockSpec((kdim, n), lambda i: (0, 0))],
        out_specs=pl.BlockSpec((BLK, n), lambda i: (i, 0)),
        out_shape=jax.ShapeDtypeStruct((mp, n), jnp.float32),
    )(xpad, w)
    return out[:m]


def _mlp_body(h_ref, w0, b0, w1, b1, w2, b2, w3, b3, wf, bf, out_ref):
    h = h_ref[...]
    h = jax.nn.relu(jnp.dot(h, w0[...], preferred_element_type=jnp.float32) + b0[...])
    h = jax.nn.relu(jnp.dot(h, w1[...], preferred_element_type=jnp.float32) + b1[...])
    h = jax.nn.relu(jnp.dot(h, w2[...], preferred_element_type=jnp.float32) + b2[...])
    h = jax.nn.relu(jnp.dot(h, w3[...], preferred_element_type=jnp.float32) + b3[...])
    out_ref[...] = jnp.dot(h, wf[...], preferred_element_type=jnp.float32) + bf[...]


def _mlp(h, params):
    BLK = 2048
    n = h.shape[0]
    ws = []
    for i in range(NUM_FF_HIDDEN + 1):
        ws += [params['ff_W%d' % i], params['ff_b%d' % i].reshape(1, FF_DIM)]
    ws += [params['ff_Wf'], params['ff_bf'].reshape(1, 1)]
    wspecs = []
    for i in range(NUM_FF_HIDDEN + 1):
        wspecs += [pl.BlockSpec((FF_DIM, FF_DIM), lambda i: (0, 0)),
                   pl.BlockSpec((1, FF_DIM), lambda i: (0, 0))]
    wspecs += [pl.BlockSpec((FF_DIM, 1), lambda i: (0, 0)),
               pl.BlockSpec((1, 1), lambda i: (0, 0))]
    out = pl.pallas_call(
        _mlp_body,
        grid=(n // BLK,),
        in_specs=[pl.BlockSpec((BLK, FF_DIM), lambda i: (i, 0))] + wspecs,
        out_specs=pl.BlockSpec((BLK, 1), lambda i: (i, 0)),
        out_shape=jax.ShapeDtypeStruct((n, 1), jnp.float32),
    )(h, *ws)
    return out.reshape(-1)


def _bn_relu(x, gamma, beta):
    mu = jnp.mean(x, axis=0)
    var = jnp.var(x, axis=0)
    return jax.nn.relu((x - mu) / jnp.sqrt(var + 1e-5) * gamma + beta)


def kernel(x_gene, x_disease, edge_index_g2d, edge_index_d2g, edge_label_index, params):
    src_g2d, dst_g2d = edge_index_g2d[0], edge_index_g2d[1]
    src_d2g, dst_d2g = edge_index_d2g[0], edge_index_d2g[1]

    hg, hd = x_gene, x_disease
    for b in range(NUM_BLOCKS):
        p = params['block%d' % b]
        gproj = _mm(hg, jnp.concatenate([p['g2d_Wl'], p['d2g_Wr']], axis=1))
        dproj = _mm(hd, jnp.concatenate([p['g2d_Wr'], p['d2g_Wl']], axis=1))
        xl_g2d, xr_d2g = gproj[:, :HIDDEN], gproj[:, HIDDEN:]
        xr_g2d, xl_d2g = dproj[:, :HIDDEN], dproj[:, HIDDEN:]
        new_d = _gat_edge_stage(xl_g2d, xr_g2d, src_g2d, dst_g2d,
                                p['g2d_att'], p['g2d_bias'])
        new_g = _gat_edge_stage(xl_d2g, xr_d2g, src_d2g, dst_d2g,
                                p['d2g_att'], p['d2g_bias'])
        hg = _bn_relu(new_g, p['bn_gene_gamma'], p['bn_gene_beta'])
        hd = _bn_relu(new_d, p['bn_disease_gamma'], p['bn_disease_beta'])

    gene_embs = hg[edge_label_index[1]]
    dis_embs = hd[edge_label_index[0]]
    h = jnp.concatenate([gene_embs, dis_embs], axis=1)
    return _mlp(h, params)
